# Initial kernel scaffold; baseline (speedup 1.0000x reference)
import jax
import jax.numpy as jnp
from jax.experimental import pallas as pl


def kernel(input_tokens, W, b):
    raise NotImplementedError("write your pallas kernel here")


# speedup vs baseline: 11.9928x; 11.9928x over previous
import functools

import jax
import jax.numpy as jnp
from jax import lax
from jax.experimental import pallas as pl
from jax.experimental.pallas import tpu as pltpu


def _probs_body(x_ref, w_ref, b_ref, probs_ref):
    x = x_ref[...]
    w = w_ref[...]
    logits = lax.dot_general(
        x, w, (((1,), (1,)), ((), ())),
        preferred_element_type=jnp.float32,
    ) + b_ref[...]
    probs_ref[...] = jax.nn.softmax(logits, axis=-1)


def _router_probs(x, W, b, block_m=512):
    M, D = x.shape
    E = W.shape[0]
    grid = (M // block_m,)
    return pl.pallas_call(
        _probs_body,
        grid=grid,
        in_specs=[
            pl.BlockSpec((block_m, D), lambda i: (i, 0)),
            pl.BlockSpec((E, D), lambda i: (0, 0)),
            pl.BlockSpec((1, E), lambda i: (0, 0)),
        ],
        out_specs=pl.BlockSpec((block_m, E), lambda i: (i, 0)),
        out_shape=jax.ShapeDtypeStruct((M, E), jnp.float32),
    )(x, W, b.reshape(1, E))


def _route_body(keys_ref, tmask_ref, gath_ref, alive_ref, *, batch, n, num_e, cap):
    alive_ref[...] = jnp.ones((batch, n), jnp.int32)
    tmask_ref[...] = jnp.zeros((batch, n), jnp.int32)
    gath_ref[...] = jnp.zeros((batch, n), jnp.float32)
    ikey = (n - 1) - lax.broadcasted_iota(jnp.int32, (batch, n), 1)
    idx_bits = (n - 1).bit_length()

    def round_fn(m, carry):
        j = (num_e - 1) - m
        k_orig = keys_ref[j]
        alive = alive_ref[...] != 0
        ke = jnp.where(alive, k_orig, jnp.int32(-1))

        th = jnp.zeros((batch, 1), jnp.int32)
        tl = jnp.zeros((batch, 1), jnp.int32)
        for bit in range(30 + idx_bits, -1, -1):
            if bit >= idx_bits:
                ch = th | jnp.int32(1 << (bit - idx_bits))
                cl = tl
            else:
                ch = th
                cl = tl | jnp.int32(1 << bit)
            pred = (ke > ch) | ((ke == ch) & (ikey >= cl))
            cnt = jnp.sum(pred.astype(jnp.int32), axis=1, keepdims=True)
            keep = cnt >= cap
            th = jnp.where(keep, ch, th)
            tl = jnp.where(keep, cl, tl)

        assign = (ke > th) | ((ke == th) & (ikey >= tl))
        tmask_ref[...] = jnp.where(assign, j, tmask_ref[...])
        gath_ref[...] = jnp.where(
            assign, lax.bitcast_convert_type(k_orig, jnp.float32), gath_ref[...])
        alive_ref[...] = jnp.where(assign, 0, alive_ref[...])
        return carry

    lax.fori_loop(0, num_e, round_fn, 0)


def _route(keysT, batch, n, num_e, cap):
    body = functools.partial(_route_body, batch=batch, n=n, num_e=num_e, cap=cap)
    return pl.pallas_call(
        body,
        out_shape=(
            jax.ShapeDtypeStruct((batch, n), jnp.int32),
            jax.ShapeDtypeStruct((batch, n), jnp.float32),
        ),
        scratch_shapes=[pltpu.VMEM((batch, n), jnp.int32)],
    )(keysT)


def kernel(input_tokens, W, b):
    batch, n, d = input_tokens.shape
    num_e = W.shape[0]
    cap = int(0.015625 * n)

    x = input_tokens.reshape(batch * n, d)
    probs = _router_probs(x, W, b)
    probsT = probs.reshape(batch, n, num_e).transpose(2, 0, 1)
    keysT = lax.bitcast_convert_type(probsT, jnp.int32)
    token_mask, gathered = _route(keysT, batch, n, num_e, cap)
    return token_mask, gathered


# speedup vs baseline: 13.6891x; 1.1414x over previous
import functools

import jax
import jax.numpy as jnp
from jax import lax
from jax.experimental import pallas as pl
from jax.experimental.pallas import tpu as pltpu

_IDX_BITS = 11


def _probs_body(x_ref, w_ref, b_ref, probs_ref):
    x = x_ref[...]
    w = w_ref[...]
    logits = lax.dot_general(
        x, w, (((1,), (1,)), ((), ())),
        preferred_element_type=jnp.float32,
    ) + b_ref[...]
    probs_ref[...] = jax.nn.softmax(logits, axis=-1)


def _router_probs(x, W, b, block_m=512):
    M, D = x.shape
    E = W.shape[0]
    grid = (M // block_m,)
    return pl.pallas_call(
        _probs_body,
        grid=grid,
        in_specs=[
            pl.BlockSpec((block_m, D), lambda i: (i, 0)),
            pl.BlockSpec((E, D), lambda i: (0, 0)),
            pl.BlockSpec((1, E), lambda i: (0, 0)),
        ],
        out_specs=pl.BlockSpec((block_m, E), lambda i: (i, 0)),
        out_shape=jax.ShapeDtypeStruct((M, E), jnp.float32),
    )(x, W, b.reshape(1, E))


def _row_count(pred):
    c = jnp.sum(pred.astype(jnp.int32), axis=1, keepdims=True)
    return c + jnp.concatenate([c[4:], c[:4]], axis=0)


def _route_body(keys_ref, tmask_ref, gath_ref, alive_ref, *, rows, cols, num_e, cap):
    alive_ref[...] = jnp.ones((rows, cols), jnp.int32)
    tmask_ref[...] = jnp.zeros((rows, cols), jnp.int32)
    gath_ref[...] = jnp.zeros((rows, cols), jnp.float32)
    sub = lax.broadcasted_iota(jnp.int32, (rows, cols), 0)
    lane = lax.broadcasted_iota(jnp.int32, (rows, cols), 1)
    ikey = jnp.where(sub < rows // 2, 2 * cols - 1 - lane, cols - 1 - lane)

    total_bits = 31 + _IDX_BITS
    pairs = [(b, b - 1) for b in range(total_bits - 1, 0, -2)]

    def round_fn(m, carry):
        j = (num_e - 1) - m
        k_orig = keys_ref[j]
        alive = alive_ref[...] != 0
        ke = jnp.where(alive, k_orig, jnp.int32(-1))

        th = jnp.zeros((rows, 1), jnp.int32)
        tl = jnp.zeros((rows, 1), jnp.int32)
        for (b1, b2) in pairs:
            cands = []
            for v in (1, 2, 3):
                hi_add = ((v >> 1) << (b1 - _IDX_BITS) if b1 >= _IDX_BITS else 0) | \
                         ((v & 1) << (b2 - _IDX_BITS) if b2 >= _IDX_BITS else 0)
                lo_add = (((v >> 1) << b1) if b1 < _IDX_BITS else 0) | \
                         (((v & 1) << b2) if b2 < _IDX_BITS else 0)
                ch = th | jnp.int32(hi_add) if hi_add else th
                cl = tl | jnp.int32(lo_add) if lo_add else tl
                if b2 >= _IDX_BITS:
                    pred = ke >= ch
                else:
                    pred = (ke > ch) | ((ke == ch) & (ikey >= cl))
                cands.append((ch, cl, _row_count(pred) >= cap))
            (c1h, c1l, ok1), (c2h, c2l, ok2), (c3h, c3l, ok3) = cands
            th = jnp.where(ok3, c3h, jnp.where(ok2, c2h, jnp.where(ok1, c1h, th)))
            tl = jnp.where(ok3, c3l, jnp.where(ok2, c2l, jnp.where(ok1, c1l, tl)))

        assign = (ke > th) | ((ke == th) & (ikey >= tl))
        tmask_ref[...] = jnp.where(assign, j, tmask_ref[...])
        gath_ref[...] = jnp.where(
            assign, lax.bitcast_convert_type(k_orig, jnp.float32), gath_ref[...])
        alive_ref[...] = jnp.where(assign, 0, alive_ref[...])
        return carry

    lax.fori_loop(0, num_e, round_fn, 0)


def _route(keysT, rows, cols, num_e, cap):
    body = functools.partial(_route_body, rows=rows, cols=cols, num_e=num_e, cap=cap)
    return pl.pallas_call(
        body,
        out_shape=(
            jax.ShapeDtypeStruct((rows, cols), jnp.int32),
            jax.ShapeDtypeStruct((rows, cols), jnp.float32),
        ),
        scratch_shapes=[pltpu.VMEM((rows, cols), jnp.int32)],
    )(keysT)


def kernel(input_tokens, W, b):
    batch, n, d = input_tokens.shape
    num_e = W.shape[0]
    cap = int(0.015625 * n)
    half = n // 2

    x = input_tokens.reshape(batch * n, d)
    probs = _router_probs(x, W, b)
    probsT = probs.reshape(batch, n, num_e).transpose(2, 0, 1)
    keysT = lax.bitcast_convert_type(probsT, jnp.int32)
    kh = keysT.reshape(num_e, batch, 2, half)
    keys8 = jnp.concatenate([kh[:, :, 0, :], kh[:, :, 1, :]], axis=1)

    tm8, g8 = _route(keys8, 2 * batch, half, num_e, cap)
    token_mask = jnp.concatenate([tm8[:batch], tm8[batch:]], axis=1)
    gathered = jnp.concatenate([g8[:batch], g8[batch:]], axis=1)
    return token_mask, gathered


# speedup vs baseline: 19.9485x; 1.4573x over previous
import functools

import jax
import jax.numpy as jnp
from jax import lax
from jax.experimental import pallas as pl
from jax.experimental.pallas import tpu as pltpu

_IDX_BITS = 11


def _probs_body(x_ref, w_ref, b_ref, probs_ref):
    x = x_ref[...]
    w = w_ref[...]
    logits = lax.dot_general(
        x, w, (((1,), (1,)), ((), ())),
        preferred_element_type=jnp.float32,
    ) + b_ref[...]
    probs_ref[...] = jax.nn.softmax(logits, axis=-1)


def _router_probs(x, W, b, block_m=512):
    M, D = x.shape
    E = W.shape[0]
    grid = (M // block_m,)
    return pl.pallas_call(
        _probs_body,
        grid=grid,
        in_specs=[
            pl.BlockSpec((block_m, D), lambda i: (i, 0)),
            pl.BlockSpec((E, D), lambda i: (0, 0)),
            pl.BlockSpec((1, E), lambda i: (0, 0)),
        ],
        out_specs=pl.BlockSpec((block_m, E), lambda i: (i, 0)),
        out_shape=jax.ShapeDtypeStruct((M, E), jnp.float32),
    )(x, W, b.reshape(1, E))


def _route_body(keys_ref, tmask_ref, gath_ref, alive_ref, *, batch, num_e, cap):
    rows, cols = 8, 256
    full = jnp.ones((batch * rows, cols), jnp.int32)
    alive_ref[...] = full
    tmask_ref[...] = jnp.zeros_like(full)
    gath_ref[...] = jnp.zeros((batch * rows, cols), jnp.float32)
    sub = lax.broadcasted_iota(jnp.int32, (rows, cols), 0)
    lane = lax.broadcasted_iota(jnp.int32, (rows, cols), 1)

    ikey = (rows * cols - 1) - (sub * cols + lane)

    total_bits = 31 + _IDX_BITS
    chunk = 4
    groups = [tuple(range(b, max(b - chunk, -1), -1))
              for b in range(total_bits - 1, 0, -chunk)]

    def round_fn(m, carry):
        j = (num_e - 1) - m
        k_all = keys_ref[j]
        alive_all = alive_ref[...]

        kb_orig, ke = [], []
        for b in range(batch):
            ko = k_all[rows * b:rows * (b + 1)]
            kb_orig.append(ko)
            ke.append(jnp.where(alive_all[rows * b:rows * (b + 1)] != 0,
                                ko, jnp.int32(-1)))

        def count(pred):
            c = jnp.sum(pred, axis=1, keepdims=True)
            c = c + pltpu.roll(c, 4, 0)
            c = c + pltpu.roll(c, 2, 0)
            c = c + pltpu.roll(c, 1, 0)
            return jnp.sum(c[0:1])

        zero = jnp.zeros((), jnp.int32)
        th = [zero] * batch
        tl = [zero] * batch
        for bits in groups:
            adds = []
            for v in range(1, 1 << len(bits)):
                hi_add, lo_add = 0, 0
                for i, bpos in enumerate(bits):
                    if not (v >> (len(bits) - 1 - i)) & 1:
                        continue
                    if bpos >= _IDX_BITS:
                        hi_add |= 1 << (bpos - _IDX_BITS)
                    else:
                        lo_add |= 1 << bpos
                adds.append((hi_add, lo_add))
            hi_only = min(bits) >= _IDX_BITS
            for b in range(batch):
                oks, chs, cls = [], [], []
                for (hi_add, lo_add) in adds:
                    ch = th[b] | jnp.int32(hi_add) if hi_add else th[b]
                    cl = tl[b] | jnp.int32(lo_add) if lo_add else tl[b]
                    if hi_only:
                        pred = ke[b] >= ch
                    else:
                        pred = (ke[b] > ch) | ((ke[b] == ch) & (ikey >= cl))
                    oks.append(count(pred) >= cap)
                    chs.append(ch)
                    cls.append(cl)
                new_th, new_tl = th[b], tl[b]
                for (ok, chv, clv) in zip(oks, chs, cls):
                    new_th = jnp.where(ok, chv, new_th)
                    new_tl = jnp.where(ok, clv, new_tl)
                th[b] = new_th
                tl[b] = new_tl

        for b in range(batch):
            sl = slice(rows * b, rows * (b + 1))
            assign = (ke[b] > th[b]) | ((ke[b] == th[b]) & (ikey >= tl[b]))
            tmask_ref[sl] = jnp.where(assign, j, tmask_ref[sl])
            gath_ref[sl] = jnp.where(
                assign, lax.bitcast_convert_type(kb_orig[b], jnp.float32),
                gath_ref[sl])
            alive_ref[sl] = jnp.where(assign, 0, alive_ref[sl])
        return carry

    lax.fori_loop(0, num_e, round_fn, 0)


def _route(keysT, batch, num_e, cap):
    body = functools.partial(_route_body, batch=batch, num_e=num_e, cap=cap)
    return pl.pallas_call(
        body,
        out_shape=(
            jax.ShapeDtypeStruct((batch * 8, 256), jnp.int32),
            jax.ShapeDtypeStruct((batch * 8, 256), jnp.float32),
        ),
        scratch_shapes=[pltpu.VMEM((batch * 8, 256), jnp.int32)],
    )(keysT)


def kernel(input_tokens, W, b):
    batch, n, d = input_tokens.shape
    num_e = W.shape[0]
    cap = int(0.015625 * n)

    x = input_tokens.reshape(batch * n, d)
    probs = _router_probs(x, W, b)
    probsT = probs.reshape(batch, n, num_e).transpose(2, 0, 1)
    keysT = lax.bitcast_convert_type(probsT, jnp.int32)
    keys32 = keysT.reshape(num_e, batch * 8, 256)
    tm, g = _route(keys32, batch, num_e, cap)
    token_mask = tm.reshape(batch, n)
    gathered = g.reshape(batch, n)
    return token_mask, gathered


# speedup vs baseline: 21.0678x; 1.0561x over previous
import functools

import jax
import jax.numpy as jnp
from jax import lax
from jax.experimental import pallas as pl
from jax.experimental.pallas import tpu as pltpu

_IDX_BITS = 11


def _probs_body(x_ref, w_ref, b_ref, keys_ref):
    lt = lax.dot_general(
        w_ref[...], x_ref[...], (((1,), (1,)), ((), ())),
        preferred_element_type=jnp.float32,
    ) + b_ref[...]
    m = jnp.max(lt, axis=0, keepdims=True)
    e = jnp.exp(lt - m)
    p = e / jnp.sum(e, axis=0, keepdims=True)
    k = lax.bitcast_convert_type(p, jnp.int32)
    n_sub = lt.shape[1] // 256
    for s in range(n_sub):
        keys_ref[:, s, :] = k[:, 256 * s:256 * (s + 1)]


def _router_keys(x, W, b, block_m=2048):
    M, D = x.shape
    E = W.shape[0]
    grid = (M // block_m,)
    n_sub = block_m // 256
    return pl.pallas_call(
        _probs_body,
        grid=grid,
        in_specs=[
            pl.BlockSpec((block_m, D), lambda i: (i, 0)),
            pl.BlockSpec((E, D), lambda i: (0, 0)),
            pl.BlockSpec((E, 1), lambda i: (0, 0)),
        ],
        out_specs=pl.BlockSpec((E, n_sub, 256), lambda i: (0, i, 0)),
        out_shape=jax.ShapeDtypeStruct((E, M // 256, 256), jnp.int32),
    )(x, W, b.reshape(E, 1))


def _route_body(keys_ref, tmask_ref, gath_ref, alive_ref, *, batch, num_e, cap):
    rows, cols = 8, 256
    full = jnp.ones((batch * rows, cols), jnp.int32)
    alive_ref[...] = full
    tmask_ref[...] = jnp.zeros_like(full)
    gath_ref[...] = jnp.zeros((batch * rows, cols), jnp.float32)
    sub = lax.broadcasted_iota(jnp.int32, (rows, cols), 0)
    lane = lax.broadcasted_iota(jnp.int32, (rows, cols), 1)

    ikey = (rows * cols - 1) - (sub * cols + lane)

    total_bits = 31 + _IDX_BITS
    chunk = 4
    groups = [tuple(range(b, max(b - chunk, -1), -1))
              for b in range(total_bits - 1, 0, -chunk)]

    def round_fn(m, carry):
        j = (num_e - 1) - m
        k_all = keys_ref[j]
        alive_all = alive_ref[...]

        kb_orig, ke = [], []
        for b in range(batch):
            ko = k_all[rows * b:rows * (b + 1)]
            kb_orig.append(ko)
            ke.append(jnp.where(alive_all[rows * b:rows * (b + 1)] != 0,
                                ko, jnp.int32(-1)))

        def count(pred):
            c = jnp.sum(pred, axis=1, keepdims=True)
            c = c + pltpu.roll(c, 4, 0)
            c = c + pltpu.roll(c, 2, 0)
            c = c + pltpu.roll(c, 1, 0)
            return jnp.sum(c[0:1])

        zero = jnp.zeros((), jnp.int32)
        th = [zero] * batch
        tl = [zero] * batch
        for bits in groups:
            adds = []
            for v in range(1, 1 << len(bits)):
                hi_add, lo_add = 0, 0
                for i, bpos in enumerate(bits):
                    if not (v >> (len(bits) - 1 - i)) & 1:
                        continue
                    if bpos >= _IDX_BITS:
                        hi_add |= 1 << (bpos - _IDX_BITS)
                    else:
                        lo_add |= 1 << bpos
                adds.append((hi_add, lo_add))
            hi_only = min(bits) >= _IDX_BITS
            for b in range(batch):
                oks, chs, cls = [], [], []
                for (hi_add, lo_add) in adds:
                    ch = th[b] | jnp.int32(hi_add) if hi_add else th[b]
                    cl = tl[b] | jnp.int32(lo_add) if lo_add else tl[b]
                    if hi_only:
                        pred = ke[b] >= ch
                    else:
                        pred = (ke[b] > ch) | ((ke[b] == ch) & (ikey >= cl))
                    oks.append(count(pred) >= cap)
                    chs.append(ch)
                    cls.append(cl)
                new_th, new_tl = th[b], tl[b]
                for (ok, chv, clv) in zip(oks, chs, cls):
                    new_th = jnp.where(ok, chv, new_th)
                    new_tl = jnp.where(ok, clv, new_tl)
                th[b] = new_th
                tl[b] = new_tl

        for b in range(batch):
            sl = slice(rows * b, rows * (b + 1))
            assign = (ke[b] > th[b]) | ((ke[b] == th[b]) & (ikey >= tl[b]))
            tmask_ref[sl] = jnp.where(assign, j, tmask_ref[sl])
            gath_ref[sl] = jnp.where(
                assign, lax.bitcast_convert_type(kb_orig[b], jnp.float32),
                gath_ref[sl])
            alive_ref[sl] = jnp.where(assign, 0, alive_ref[sl])
        return carry

    lax.fori_loop(0, num_e, round_fn, 0)


def _route(keysT, batch, num_e, cap):
    body = functools.partial(_route_body, batch=batch, num_e=num_e, cap=cap)
    return pl.pallas_call(
        body,
        out_shape=(
            jax.ShapeDtypeStruct((batch * 8, 256), jnp.int32),
            jax.ShapeDtypeStruct((batch * 8, 256), jnp.float32),
        ),
        scratch_shapes=[pltpu.VMEM((batch * 8, 256), jnp.int32)],
    )(keysT)


def kernel(input_tokens, W, b):
    batch, n, d = input_tokens.shape
    num_e = W.shape[0]
    cap = int(0.015625 * n)

    x = input_tokens.reshape(batch * n, d)
    keys32 = _router_keys(x, W, b)
    tm, g = _route(keys32, batch, num_e, cap)
    token_mask = tm.reshape(batch, n)
    gathered = g.reshape(batch, n)
    return token_mask, gathered


# speedup vs baseline: 22.1278x; 1.0503x over previous
import functools

import jax
import jax.numpy as jnp
from jax import lax
from jax.experimental import pallas as pl
from jax.experimental.pallas import tpu as pltpu

_IDX_BITS = 11


def _probs_body(x_ref, w_ref, b_ref, keys_ref):
    lt = lax.dot_general(
        w_ref[...], x_ref[...], (((1,), (1,)), ((), ())),
        preferred_element_type=jnp.float32,
    ) + b_ref[...]
    m = jnp.max(lt, axis=0, keepdims=True)
    e = jnp.exp(lt - m)
    p = e / jnp.sum(e, axis=0, keepdims=True)
    k = lax.bitcast_convert_type(p, jnp.int32)
    n_sub = lt.shape[1] // 256
    for s in range(n_sub):
        keys_ref[:, s, :] = k[:, 256 * s:256 * (s + 1)]


def _router_keys(x, W, b, block_m=2048):
    M, D = x.shape
    E = W.shape[0]
    grid = (M // block_m,)
    n_sub = block_m // 256
    return pl.pallas_call(
        _probs_body,
        grid=grid,
        in_specs=[
            pl.BlockSpec((block_m, D), lambda i: (i, 0)),
            pl.BlockSpec((E, D), lambda i: (0, 0)),
            pl.BlockSpec((E, 1), lambda i: (0, 0)),
        ],
        out_specs=pl.BlockSpec((E, n_sub, 256), lambda i: (0, i, 0)),
        out_shape=jax.ShapeDtypeStruct((E, M // 256, 256), jnp.int32),
    )(x, W, b.reshape(E, 1))


def _route_body(keys_ref, tmask_ref, gath_ref, alive_ref, *, batch, num_e, cap):
    rows, cols = 8, 256
    full = jnp.ones((batch * rows, cols), jnp.int32)
    alive_ref[...] = full
    tmask_ref[...] = jnp.zeros_like(full)
    gath_ref[...] = jnp.zeros((batch * rows, cols), jnp.float32)
    sub = lax.broadcasted_iota(jnp.int32, (rows, cols), 0)
    lane = lax.broadcasted_iota(jnp.int32, (rows, cols), 1)

    ikey = (rows * cols - 1) - (sub * cols + lane)

    total_bits = 31 + _IDX_BITS
    chunk = 4
    groups = [tuple(range(b, max(b - chunk, -1), -1))
              for b in range(total_bits - 1, 0, -chunk)]

    def one_round(j):
        k_all = keys_ref[j]
        alive_all = alive_ref[...]

        kb_orig, ke = [], []
        for b in range(batch):
            ko = k_all[rows * b:rows * (b + 1)]
            kb_orig.append(ko)
            ke.append(jnp.where(alive_all[rows * b:rows * (b + 1)] != 0,
                                ko, jnp.int32(-1)))

        def count(pred):
            c = jnp.sum(pred, axis=1, keepdims=True)
            c = c + pltpu.roll(c, 4, 0)
            c = c + pltpu.roll(c, 2, 0)
            c = c + pltpu.roll(c, 1, 0)
            return jnp.sum(c[0:1])

        zero = jnp.zeros((), jnp.int32)
        th = [zero] * batch
        tl = [zero] * batch
        for bits in groups:
            adds = []
            for v in range(1, 1 << len(bits)):
                hi_add, lo_add = 0, 0
                for i, bpos in enumerate(bits):
                    if not (v >> (len(bits) - 1 - i)) & 1:
                        continue
                    if bpos >= _IDX_BITS:
                        hi_add |= 1 << (bpos - _IDX_BITS)
                    else:
                        lo_add |= 1 << bpos
                adds.append((hi_add, lo_add))
            hi_only = min(bits) >= _IDX_BITS
            for b in range(batch):
                oks, chs, cls = [], [], []
                for (hi_add, lo_add) in adds:
                    ch = th[b] | jnp.int32(hi_add) if hi_add else th[b]
                    cl = tl[b] | jnp.int32(lo_add) if lo_add else tl[b]
                    if hi_only:
                        pred = ke[b] >= ch
                    else:
                        pred = (ke[b] > ch) | ((ke[b] == ch) & (ikey >= cl))
                    oks.append(count(pred) >= cap)
                    chs.append(ch)
                    cls.append(cl)
                new_th, new_tl = th[b], tl[b]
                for (ok, chv, clv) in zip(oks, chs, cls):
                    new_th = jnp.where(ok, chv, new_th)
                    new_tl = jnp.where(ok, clv, new_tl)
                th[b] = new_th
                tl[b] = new_tl

        for b in range(batch):
            sl = slice(rows * b, rows * (b + 1))
            assign = (ke[b] > th[b]) | ((ke[b] == th[b]) & (ikey >= tl[b]))
            tmask_ref[sl] = jnp.where(assign, j, tmask_ref[sl])
            gath_ref[sl] = jnp.where(
                assign, lax.bitcast_convert_type(kb_orig[b], jnp.float32),
                gath_ref[sl])
            alive_ref[sl] = jnp.where(assign, 0, alive_ref[sl])

    def round_fn(m, carry):
        one_round((num_e - 1) - 2 * m)
        one_round((num_e - 2) - 2 * m)
        return carry

    lax.fori_loop(0, num_e // 2, round_fn, 0)


def _route(keysT, batch, num_e, cap):
    body = functools.partial(_route_body, batch=batch, num_e=num_e, cap=cap)
    return pl.pallas_call(
        body,
        out_shape=(
            jax.ShapeDtypeStruct((batch * 8, 256), jnp.int32),
            jax.ShapeDtypeStruct((batch * 8, 256), jnp.float32),
        ),
        scratch_shapes=[pltpu.VMEM((batch * 8, 256), jnp.int32)],
    )(keysT)


def kernel(input_tokens, W, b):
    batch, n, d = input_tokens.shape
    num_e = W.shape[0]
    cap = int(0.015625 * n)

    x = input_tokens.reshape(batch * n, d)
    keys32 = _router_keys(x, W, b)
    tm, g = _route(keys32, batch, num_e, cap)
    token_mask = tm.reshape(batch, n)
    gathered = g.reshape(batch, n)
    return token_mask, gathered


# speedup vs baseline: 22.4844x; 1.0161x over previous
import functools

import jax
import jax.numpy as jnp
from jax import lax
from jax.experimental import pallas as pl
from jax.experimental.pallas import tpu as pltpu

_IDX_BITS = 11


def _probs_body(x_ref, w_ref, b_ref, keys_ref):
    lt = lax.dot_general(
        w_ref[...], x_ref[...], (((1,), (1,)), ((), ())),
        preferred_element_type=jnp.float32,
    ) + b_ref[...]
    m = jnp.max(lt, axis=0, keepdims=True)
    e = jnp.exp(lt - m)
    p = e / jnp.sum(e, axis=0, keepdims=True)
    k = lax.bitcast_convert_type(p, jnp.int32)
    n_sub = lt.shape[1] // 256
    for s in range(n_sub):
        keys_ref[:, s, :] = k[:, 256 * s:256 * (s + 1)]


def _router_keys(x, W, b, block_m=2048):
    M, D = x.shape
    E = W.shape[0]
    grid = (M // block_m,)
    n_sub = block_m // 256
    return pl.pallas_call(
        _probs_body,
        grid=grid,
        in_specs=[
            pl.BlockSpec((block_m, D), lambda i: (i, 0)),
            pl.BlockSpec((E, D), lambda i: (0, 0)),
            pl.BlockSpec((E, 1), lambda i: (0, 0)),
        ],
        out_specs=pl.BlockSpec((E, n_sub, 256), lambda i: (0, i, 0)),
        out_shape=jax.ShapeDtypeStruct((E, M // 256, 256), jnp.int32),
    )(x, W, b.reshape(E, 1))


def _route_body(keys_ref, tmask_ref, gath_ref, alive_ref, *, batch, num_e, cap):
    rows, cols = 8, 256
    full = jnp.ones((batch * rows, cols), jnp.int32)
    alive_ref[...] = full
    tmask_ref[...] = jnp.zeros_like(full)
    gath_ref[...] = jnp.zeros((batch * rows, cols), jnp.float32)
    sub = lax.broadcasted_iota(jnp.int32, (rows, cols), 0)
    lane = lax.broadcasted_iota(jnp.int32, (rows, cols), 1)

    ikey = (rows * cols - 1) - (sub * cols + lane)

    total_bits = 31 + _IDX_BITS
    chunk = 4
    groups = [tuple(range(b, max(b - chunk, -1), -1))
              for b in range(total_bits - 1, 0, -chunk)]

    def one_round(j):
        k_all = keys_ref[j]
        alive_all = alive_ref[...]

        kb_orig, ke = [], []
        for b in range(batch):
            ko = k_all[rows * b:rows * (b + 1)]
            kb_orig.append(ko)
            ke.append(jnp.where(alive_all[rows * b:rows * (b + 1)] != 0,
                                ko, jnp.int32(-1)))

        def count(pred):
            c = jnp.sum(pred, axis=1, keepdims=True)
            c = c + pltpu.roll(c, 4, 0)
            c = c + pltpu.roll(c, 2, 0)
            c = c + pltpu.roll(c, 1, 0)
            return jnp.sum(c[0:1])

        zero = jnp.zeros((), jnp.int32)
        th = [zero] * batch
        tl = [zero] * batch
        for bits in groups:
            adds = []
            for v in range(1, 1 << len(bits)):
                hi_add, lo_add = 0, 0
                for i, bpos in enumerate(bits):
                    if not (v >> (len(bits) - 1 - i)) & 1:
                        continue
                    if bpos >= _IDX_BITS:
                        hi_add |= 1 << (bpos - _IDX_BITS)
                    else:
                        lo_add |= 1 << bpos
                adds.append((hi_add, lo_add))
            hi_only = min(bits) >= _IDX_BITS
            for b in range(batch):
                oks, chs, cls = [], [], []
                for (hi_add, lo_add) in adds:
                    ch = th[b] | jnp.int32(hi_add) if hi_add else th[b]
                    cl = tl[b] | jnp.int32(lo_add) if lo_add else tl[b]
                    if hi_only:
                        pred = ke[b] >= ch
                    else:
                        pred = (ke[b] > ch) | ((ke[b] == ch) & (ikey >= cl))
                    oks.append(count(pred) >= cap)
                    chs.append(ch)
                    cls.append(cl)
                new_th, new_tl = th[b], tl[b]
                for (ok, chv, clv) in zip(oks, chs, cls):
                    new_th = jnp.where(ok, chv, new_th)
                    new_tl = jnp.where(ok, clv, new_tl)
                th[b] = new_th
                tl[b] = new_tl

        for b in range(batch):
            sl = slice(rows * b, rows * (b + 1))
            assign = (ke[b] > th[b]) | ((ke[b] == th[b]) & (ikey >= tl[b]))
            tmask_ref[sl] = jnp.where(assign, j, tmask_ref[sl])
            gath_ref[sl] = jnp.where(
                assign, lax.bitcast_convert_type(kb_orig[b], jnp.float32),
                gath_ref[sl])
            alive_ref[sl] = jnp.where(assign, 0, alive_ref[sl])

    def round_fn(m, carry):
        for u in range(4):
            one_round((num_e - 1 - u) - 4 * m)
        return carry

    lax.fori_loop(0, num_e // 4, round_fn, 0)


def _route(keysT, batch, num_e, cap):
    body = functools.partial(_route_body, batch=batch, num_e=num_e, cap=cap)
    return pl.pallas_call(
        body,
        out_shape=(
            jax.ShapeDtypeStruct((batch * 8, 256), jnp.int32),
            jax.ShapeDtypeStruct((batch * 8, 256), jnp.float32),
        ),
        scratch_shapes=[pltpu.VMEM((batch * 8, 256), jnp.int32)],
    )(keysT)


def kernel(input_tokens, W, b):
    batch, n, d = input_tokens.shape
    num_e = W.shape[0]
    cap = int(0.015625 * n)

    x = input_tokens.reshape(batch * n, d)
    keys32 = _router_keys(x, W, b)
    tm, g = _route(keys32, batch, num_e, cap)
    token_mask = tm.reshape(batch, n)
    gathered = g.reshape(batch, n)
    return token_mask, gathered
